# Initial kernel scaffold; baseline (speedup 1.0000x reference)
import jax
import jax.numpy as jnp
from jax.experimental import pallas as pl


def kernel(x, edge_index, W1l, b1, W1r, gamma, beta, run_mean, run_var, W2l, b2, W2r):
    raise NotImplementedError("write your pallas kernel here")


# speedup vs baseline: 8.5273x; 8.5273x over previous
import functools

import jax
import jax.numpy as jnp
from jax import lax
from jax.experimental import pallas as pl
from jax.experimental.pallas import tpu as pltpu
from jax.experimental.pallas import tpu_sc as plsc

N = 10000
E = 320000
D = 128
BN_EPS = 1e-5

NC = 2
NS = 16
NW = NC * NS
EPW = E // NW
CHUNK = 125
NCHUNK = EPW // CHUNK
N_PAD = 10240
ROWS_PER_TILE = N_PAD // NS
RB = 128
CNT_PER_TILE = N_PAD // NS


def _zero_vec16(ref, nwords):
    def body(k, _):
        ref[pl.ds(k * 16, 16)] = jnp.zeros((16,), jnp.float32)
        return 0
    lax.fori_loop(0, nwords // 16, body, 0)


def _sc_agg_body(p_hbm, src_hbm, dst_hbm, out_acc, out_cnt,
                 src_v, dst_v, rows_v, cbuf, ones_v, acc_sh, cnt_sh):
    cid = lax.axis_index("c")
    sid = lax.axis_index("s")
    wid = cid * NS + sid

    def zrow(k, _):
        r = k // (D // 16)
        c = (k % (D // 16)) * 16
        rows_v[r, pl.ds(c, 16)] = jnp.zeros((16,), jnp.float32)
        return 0
    lax.fori_loop(0, RB * (D // 16), zrow, 0)
    _zero_vec16(cbuf, CNT_PER_TILE)

    def orow(k, _):
        ones_v[pl.ds(k * 16, 16)] = jnp.ones((16,), jnp.float32)
        return 0
    lax.fori_loop(0, 8, orow, 0)

    for k in range(ROWS_PER_TILE // RB):
        pltpu.sync_copy(rows_v, acc_sh.at[pl.ds(sid * ROWS_PER_TILE + k * RB, RB)])
    pltpu.sync_copy(cbuf, cnt_sh.at[pl.ds(sid * CNT_PER_TILE, CNT_PER_TILE)])
    plsc.subcore_barrier()

    pltpu.sync_copy(src_hbm.at[wid], src_v)
    pltpu.sync_copy(dst_hbm.at[wid], dst_v)

    def chunk_body(j, _):
        pltpu.sync_copy(p_hbm.at[src_v.at[j]], rows_v.at[pl.ds(0, CHUNK)])
        pltpu.sync_copy(rows_v.at[pl.ds(0, CHUNK)], acc_sh.at[dst_v.at[j]], add=True)
        pltpu.sync_copy(ones_v.at[pl.ds(0, CHUNK)], cnt_sh.at[dst_v.at[j]], add=True)
        return 0
    lax.fori_loop(0, NCHUNK, chunk_body, 0)
    plsc.subcore_barrier()

    for k in range(ROWS_PER_TILE // RB):
        r0 = sid * ROWS_PER_TILE + k * RB
        pltpu.sync_copy(acc_sh.at[pl.ds(r0, RB)], rows_v)
        pltpu.sync_copy(rows_v, out_acc.at[cid, pl.ds(r0, RB)])
    c0 = sid * CNT_PER_TILE
    pltpu.sync_copy(cnt_sh.at[pl.ds(c0, CNT_PER_TILE)], cbuf)
    pltpu.sync_copy(cbuf, out_cnt.at[cid, pl.ds(c0, CNT_PER_TILE)])


_sc_aggregate = functools.partial(
    pl.kernel,
    out_type=(
        jax.ShapeDtypeStruct((NC, N_PAD, D), jnp.float32),
        jax.ShapeDtypeStruct((NC, N_PAD), jnp.float32),
    ),
    mesh=plsc.VectorSubcoreMesh(
        core_axis_name="c", subcore_axis_name="s", num_cores=NC, num_subcores=NS
    ),
    scratch_types=[
        pltpu.VMEM((NCHUNK, CHUNK), jnp.int32),
        pltpu.VMEM((NCHUNK, CHUNK), jnp.int32),
        pltpu.VMEM((RB, D), jnp.float32),
        pltpu.VMEM((CNT_PER_TILE,), jnp.float32),
        pltpu.VMEM((128,), jnp.float32),
        pltpu.VMEM_SHARED((N_PAD, D), jnp.float32),
        pltpu.VMEM_SHARED((N_PAD,), jnp.float32),
    ],
)(_sc_agg_body)


BLK = 1024
GRID = (N + BLK - 1) // BLK


def _proj_body(x_ref, w_ref, a_ref, b_ref):
    xw = jnp.dot(x_ref[...], w_ref[...], preferred_element_type=jnp.float32)
    a_ref[...] = xw[:, :D]
    b_ref[...] = xw[:, D:]


def _project(x, wcat):
    return pl.pallas_call(
        _proj_body,
        grid=(GRID,),
        in_specs=[
            pl.BlockSpec((BLK, D), lambda i: (i, 0)),
            pl.BlockSpec((D, 2 * D), lambda i: (0, 0)),
        ],
        out_specs=[
            pl.BlockSpec((BLK, D), lambda i: (i, 0)),
            pl.BlockSpec((BLK, D), lambda i: (i, 0)),
        ],
        out_shape=[
            jax.ShapeDtypeStruct((N, D), jnp.float32),
            jax.ShapeDtypeStruct((N, D), jnp.float32),
        ],
    )(x, wcat)


def _mid_body(sp_ref, cnt_ref, xr_ref, b1_ref, bnm_ref, bna_ref, w2_ref,
              p2_ref, hr_ref):
    s = sp_ref[0] + sp_ref[1]
    cnt = cnt_ref[0] + cnt_ref[1]
    inv = 1.0 / jnp.maximum(cnt, 1.0)
    h = s * inv + b1_ref[...] + xr_ref[...]
    h = h * bnm_ref[...] + bna_ref[...]
    h = jnp.maximum(h, 0.0)
    hw = jnp.dot(h, w2_ref[...], preferred_element_type=jnp.float32)
    p2_ref[...] = hw[:, :D]
    hr_ref[...] = hw[:, D:]


def _mid(s1p, cnt2, xr, b1r, bnm, bna, w2cat):
    return pl.pallas_call(
        _mid_body,
        grid=(GRID,),
        in_specs=[
            pl.BlockSpec((2, BLK, D), lambda i: (0, i, 0)),
            pl.BlockSpec((2, BLK, 1), lambda i: (0, i, 0)),
            pl.BlockSpec((BLK, D), lambda i: (i, 0)),
            pl.BlockSpec((1, D), lambda i: (0, 0)),
            pl.BlockSpec((1, D), lambda i: (0, 0)),
            pl.BlockSpec((1, D), lambda i: (0, 0)),
            pl.BlockSpec((D, 2 * D), lambda i: (0, 0)),
        ],
        out_specs=[
            pl.BlockSpec((BLK, D), lambda i: (i, 0)),
            pl.BlockSpec((BLK, D), lambda i: (i, 0)),
        ],
        out_shape=[
            jax.ShapeDtypeStruct((N, D), jnp.float32),
            jax.ShapeDtypeStruct((N, D), jnp.float32),
        ],
    )(s1p, cnt2, xr, b1r, bnm, bna, w2cat)


def _final_body(sp_ref, cnt_ref, hr_ref, b2_ref, out_ref):
    s = sp_ref[0] + sp_ref[1]
    cnt = cnt_ref[0] + cnt_ref[1]
    inv = 1.0 / jnp.maximum(cnt, 1.0)
    out_ref[...] = s * inv + b2_ref[...] + hr_ref[...]


def _final(s2p, cnt2, hr, b2r):
    return pl.pallas_call(
        _final_body,
        grid=(GRID,),
        in_specs=[
            pl.BlockSpec((2, BLK, D), lambda i: (0, i, 0)),
            pl.BlockSpec((2, BLK, 1), lambda i: (0, i, 0)),
            pl.BlockSpec((BLK, D), lambda i: (i, 0)),
            pl.BlockSpec((1, D), lambda i: (0, 0)),
        ],
        out_specs=pl.BlockSpec((BLK, D), lambda i: (i, 0)),
        out_shape=jax.ShapeDtypeStruct((N, D), jnp.float32),
    )(s2p, cnt2, hr, b2r)


def kernel(x, edge_index, W1l, b1, W1r, gamma, beta, run_mean, run_var, W2l, b2, W2r):
    src3 = edge_index[0].reshape(NW, NCHUNK, CHUNK)
    dst3 = edge_index[1].reshape(NW, NCHUNK, CHUNK)

    bnm = (gamma * lax.rsqrt(run_var + BN_EPS)).reshape(1, D)
    bna = (beta - run_mean * bnm[0]).reshape(1, D)
    b1r = b1.reshape(1, D)
    b2r = b2.reshape(1, D)
    w1cat = jnp.concatenate([W1l, W1r], axis=1)
    w2cat = jnp.concatenate([W2l, W2r], axis=1)

    p1, xr = _project(x, w1cat)
    s1p, cntp = _sc_aggregate(p1, src3, dst3)
    cnt2 = cntp[:, :, None]

    p2, hr = _mid(s1p, cnt2, xr, b1r, bnm, bna, w2cat)

    s2p, _ = _sc_aggregate(p2, src3, dst3)
    return _final(s2p, cnt2, hr, b2r)
